# Initial kernel scaffold; baseline (speedup 1.0000x reference)
import jax
import jax.numpy as jnp
from jax.experimental import pallas as pl


def kernel(x, edge_index, W1, b1, W2, b2):
    raise NotImplementedError("write your pallas kernel here")


# speedup vs baseline: 10.8205x; 10.8205x over previous
import functools

import jax
import jax.numpy as jnp
from jax import lax
from jax.experimental import pallas as pl
from jax.experimental.pallas import tpu as pltpu
from jax.experimental.pallas import tpu_sc as plsc

N_NODES = 10000
D_IN = 128
D_HID = 128
D_OUT = 64
N_PAD = 10240
E = 320000
CHUNK = 128
NC, NS = 2, 16
NW = NC * NS
E_TILE = 10240
E_PAD = E_TILE * NW
NCHUNK = E_TILE // CHUNK
STRIPE = N_PAD // NS


def _mesh():
    return plsc.VectorSubcoreMesh(
        core_axis_name="c", subcore_axis_name="s", num_cores=NC, num_subcores=NS
    )


def _sc_degree(dst2d, ones_blk, zeros_stripe):
    def body(dst_hbm, ones_hbm, zeros_hbm, out_hbm, dst_v, ones_v, acc_sh):
        cid = lax.axis_index("c")
        sid = lax.axis_index("s")
        wid = cid * NS + sid
        pltpu.sync_copy(zeros_hbm, acc_sh.at[pl.ds(sid * STRIPE, STRIPE)])
        pltpu.sync_copy(ones_hbm, ones_v)
        pltpu.sync_copy(dst_hbm.at[pl.ds(wid * NCHUNK, NCHUNK)], dst_v)
        plsc.subcore_barrier()

        def step(j, carry):
            pltpu.sync_copy(ones_v, acc_sh.at[dst_v.at[j]], add=True)
            return carry

        lax.fori_loop(0, NCHUNK, step, 0)
        plsc.subcore_barrier()
        pltpu.sync_copy(
            acc_sh.at[pl.ds(sid * STRIPE, STRIPE)],
            out_hbm.at[cid].at[pl.ds(sid * STRIPE, STRIPE)],
        )

    return pl.kernel(
        body,
        out_type=jax.ShapeDtypeStruct((NC, N_PAD, 16), jnp.float32),
        mesh=_mesh(),
        compiler_params=pltpu.CompilerParams(use_tc_tiling_on_sc=False),
        scratch_types=[
            pltpu.VMEM((NCHUNK, CHUNK), jnp.int32),
            pltpu.VMEM((CHUNK, 16), jnp.float32),
            pltpu.VMEM_SHARED((N_PAD, 16), jnp.float32),
        ],
    )(dst2d, ones_blk, zeros_stripe)


def _sc_aggregate(y_pad, src1d, dst2d, zeros_stripe, d):
    def body(y_hbm, src_hbm, dst_hbm, zeros_hbm, out_hbm, src_v, dst_v, rows_v, acc_sh):
        cid = lax.axis_index("c")
        sid = lax.axis_index("s")
        wid = cid * NS + sid
        pltpu.sync_copy(zeros_hbm, acc_sh.at[pl.ds(sid * STRIPE, STRIPE)])
        pltpu.sync_copy(src_hbm.at[pl.ds(wid * E_TILE, E_TILE)], src_v)
        pltpu.sync_copy(dst_hbm.at[pl.ds(wid * NCHUNK, NCHUNK)], dst_v)
        plsc.subcore_barrier()

        def step(j, carry):
            off = pl.multiple_of(j * CHUNK, CHUNK)
            pltpu.sync_copy(y_hbm.at[src_v.at[pl.ds(off, CHUNK)]], rows_v)
            pltpu.sync_copy(rows_v, acc_sh.at[dst_v.at[j]], add=True)
            return carry

        lax.fori_loop(0, NCHUNK, step, 0)
        plsc.subcore_barrier()
        pltpu.sync_copy(acc_sh.at[pl.ds(sid * STRIPE, STRIPE)], out_hbm.at[cid].at[pl.ds(sid * STRIPE, STRIPE)])

    return pl.kernel(
        body,
        out_type=jax.ShapeDtypeStruct((NC, N_PAD, d), jnp.float32),
        mesh=_mesh(),
        compiler_params=pltpu.CompilerParams(use_tc_tiling_on_sc=False),
        scratch_types=[
            pltpu.VMEM((E_TILE,), jnp.int32),
            pltpu.VMEM((NCHUNK, CHUNK), jnp.int32),
            pltpu.VMEM((CHUNK, d), jnp.float32),
            pltpu.VMEM_SHARED((N_PAD, d), jnp.float32),
        ],
    )(y_pad, src1d, dst2d, zeros_stripe)


_ROW_BLK = 2048


def _dinv_block(deg_ref):
    deg = deg_ref[0, :, 0:1] + deg_ref[1, :, 0:1] + 1.0
    return lax.rsqrt(deg)


def _tc_y1(deg_parts, x_pad, w1):
    def body(deg_ref, x_ref, w_ref, y_ref):
        dinv = _dinv_block(deg_ref)
        y_ref[...] = (
            jnp.dot(x_ref[...], w_ref[...], preferred_element_type=jnp.float32) * dinv
        )

    return pl.pallas_call(
        body,
        grid=(N_PAD // _ROW_BLK,),
        in_specs=[
            pl.BlockSpec((NC, _ROW_BLK, 16), lambda i: (0, i, 0)),
            pl.BlockSpec((_ROW_BLK, D_IN), lambda i: (i, 0)),
            pl.BlockSpec((D_IN, D_HID), lambda i: (0, 0)),
        ],
        out_specs=pl.BlockSpec((_ROW_BLK, D_HID), lambda i: (i, 0)),
        out_shape=jax.ShapeDtypeStruct((N_PAD, D_HID), jnp.float32),
    )(deg_parts, x_pad, w1)


def _tc_h_y2(deg_parts, acc1, y1, w2, b1):
    def body(deg_ref, acc_ref, y1_ref, w_ref, b_ref, y2_ref):
        dinv = _dinv_block(deg_ref)
        tot = acc_ref[0] + acc_ref[1] + y1_ref[...]
        h = jnp.maximum(tot * dinv + b_ref[...], 0.0)
        ridx = pl.program_id(0) * _ROW_BLK + lax.broadcasted_iota(
            jnp.int32, (_ROW_BLK, 1), 0
        )
        h = jnp.where(ridx < N_NODES, h, 0.0)
        y2_ref[...] = (
            jnp.dot(h, w_ref[...], preferred_element_type=jnp.float32) * dinv
        )

    return pl.pallas_call(
        body,
        grid=(N_PAD // _ROW_BLK,),
        in_specs=[
            pl.BlockSpec((NC, _ROW_BLK, 16), lambda i: (0, i, 0)),
            pl.BlockSpec((NC, _ROW_BLK, D_HID), lambda i: (0, i, 0)),
            pl.BlockSpec((_ROW_BLK, D_HID), lambda i: (i, 0)),
            pl.BlockSpec((D_HID, D_OUT), lambda i: (0, 0)),
            pl.BlockSpec((1, D_HID), lambda i: (0, 0)),
        ],
        out_specs=pl.BlockSpec((_ROW_BLK, D_OUT), lambda i: (i, 0)),
        out_shape=jax.ShapeDtypeStruct((N_PAD, D_OUT), jnp.float32),
    )(deg_parts, acc1, y1, w2, b1)


def _tc_out(deg_parts, acc2, y2, b2):
    def body(deg_ref, acc_ref, y2_ref, b_ref, o_ref):
        dinv = _dinv_block(deg_ref)
        tot = acc_ref[0] + acc_ref[1] + y2_ref[...]
        o_ref[...] = jnp.maximum(tot * dinv + b_ref[...], 0.0)

    return pl.pallas_call(
        body,
        grid=(N_PAD // _ROW_BLK,),
        in_specs=[
            pl.BlockSpec((NC, _ROW_BLK, 16), lambda i: (0, i, 0)),
            pl.BlockSpec((NC, _ROW_BLK, D_OUT), lambda i: (0, i, 0)),
            pl.BlockSpec((_ROW_BLK, D_OUT), lambda i: (i, 0)),
            pl.BlockSpec((1, D_OUT), lambda i: (0, 0)),
        ],
        out_specs=pl.BlockSpec((_ROW_BLK, D_OUT), lambda i: (i, 0)),
        out_shape=jax.ShapeDtypeStruct((N_PAD, D_OUT), jnp.float32),
    )(deg_parts, acc2, y2, b2)


def kernel(x, edge_index, W1, b1, W2, b2):
    src = edge_index[0].astype(jnp.int32)
    dst = edge_index[1].astype(jnp.int32)
    pad = jnp.full((E_PAD - E,), N_NODES, jnp.int32)
    src_pad = jnp.concatenate([src, pad])
    dst2d = jnp.concatenate([dst, pad]).reshape(E_PAD // CHUNK, CHUNK)
    x_pad = jnp.concatenate(
        [x, jnp.zeros((N_PAD - N_NODES, x.shape[1]), x.dtype)]
    )
    ones16 = jnp.ones((CHUNK, 16), jnp.float32)
    z16 = jnp.zeros((STRIPE, 16), jnp.float32)
    z128 = jnp.zeros((STRIPE, D_HID), jnp.float32)
    z64 = jnp.zeros((STRIPE, D_OUT), jnp.float32)

    deg_parts = _sc_degree(dst2d, ones16, z16)
    y1 = _tc_y1(deg_parts, x_pad, W1)
    acc1 = _sc_aggregate(y1, src_pad, dst2d, z128, D_HID)
    y2 = _tc_h_y2(deg_parts, acc1, y1, W2, b1.reshape(1, -1))
    acc2 = _sc_aggregate(y2, src_pad, dst2d, z64, D_OUT)
    out = _tc_out(deg_parts, acc2, y2, b2.reshape(1, -1))
    return out[:N_NODES]


# speedup vs baseline: 16.5291x; 1.5276x over previous
import functools

import jax
import jax.numpy as jnp
from jax import lax
from jax.experimental import pallas as pl
from jax.experimental.pallas import tpu as pltpu
from jax.experimental.pallas import tpu_sc as plsc

N_NODES = 10000
D_IN = 128
D_HID = 128
D_OUT = 64
N_PAD = 10240
E = 320000
CHUNK = 128
NC, NS = 2, 16
NW = NC * NS
E_TILE = 10240
E_PAD = E_TILE * NW
NCHUNK = E_TILE // CHUNK
STRIPE = N_PAD // NS


def _mesh():
    return plsc.VectorSubcoreMesh(
        core_axis_name="c", subcore_axis_name="s", num_cores=NC, num_subcores=NS
    )


def _sc_degree(dst2d, ones_blk, zeros_stripe):
    def body(dst_hbm, ones_hbm, zeros_hbm, out_hbm, dst_v, ones_v, acc_sh):
        cid = lax.axis_index("c")
        sid = lax.axis_index("s")
        wid = cid * NS + sid
        pltpu.sync_copy(zeros_hbm, acc_sh.at[pl.ds(sid * STRIPE, STRIPE)])
        pltpu.sync_copy(ones_hbm, ones_v)
        pltpu.sync_copy(dst_hbm.at[pl.ds(wid * NCHUNK, NCHUNK)], dst_v)
        plsc.subcore_barrier()

        def step(j, carry):
            pltpu.sync_copy(ones_v, acc_sh.at[dst_v.at[j]], add=True)
            return carry

        lax.fori_loop(0, NCHUNK, step, 0)
        plsc.subcore_barrier()
        pltpu.sync_copy(
            acc_sh.at[pl.ds(sid * STRIPE, STRIPE)],
            out_hbm.at[cid].at[pl.ds(sid * STRIPE, STRIPE)],
        )

    return pl.kernel(
        body,
        out_type=jax.ShapeDtypeStruct((NC, N_PAD, 16), jnp.float32),
        mesh=_mesh(),
        compiler_params=pltpu.CompilerParams(use_tc_tiling_on_sc=False),
        scratch_types=[
            pltpu.VMEM((NCHUNK, CHUNK), jnp.int32),
            pltpu.VMEM((CHUNK, 16), jnp.float32),
            pltpu.VMEM_SHARED((N_PAD, 16), jnp.float32),
        ],
    )(dst2d, ones_blk, zeros_stripe)


E_TILE2 = E_PAD // NS
NCHUNK2 = E_TILE2 // CHUNK


def _sc_aggregate(y_split, src1d, dst2d, zeros_stripe, dh):
    NBUF = 4

    def body(y_hbm, src_hbm, dst_hbm, zeros_hbm, out_hbm, src_v, dst_v, rows_v, acc_sh,
             *sems):
        cid = lax.axis_index("c")
        sid = lax.axis_index("s")
        pltpu.sync_copy(zeros_hbm, acc_sh.at[pl.ds(sid * STRIPE, STRIPE)])
        pltpu.sync_copy(src_hbm.at[pl.ds(sid * E_TILE2, E_TILE2)], src_v)
        pltpu.sync_copy(dst_hbm.at[pl.ds(sid * NCHUNK2, NCHUNK2)], dst_v)
        plsc.subcore_barrier()

        def gather(j, b):
            off = pl.multiple_of(j * CHUNK, CHUNK)
            return pltpu.async_copy(
                y_hbm.at[cid].at[src_v.at[pl.ds(off, CHUNK)]], rows_v.at[b], sems[b]
            )

        for b in range(NBUF):
            gather(b, b)

        @pl.loop(0, NCHUNK2, step=NBUF)
        def _(j0):
            for b in range(NBUF):
                j = j0 + b
                pltpu.make_async_copy(
                    y_hbm.at[cid].at[
                        src_v.at[pl.ds(pl.multiple_of(j * CHUNK, CHUNK), CHUNK)]
                    ],
                    rows_v.at[b],
                    sems[b],
                ).wait()
                pltpu.sync_copy(rows_v.at[b], acc_sh.at[dst_v.at[j]], add=True)

                @pl.when(j + NBUF < NCHUNK2)
                def _():
                    gather(j + NBUF, b)

        plsc.subcore_barrier()
        pltpu.sync_copy(acc_sh.at[pl.ds(sid * STRIPE, STRIPE)], out_hbm.at[cid].at[pl.ds(sid * STRIPE, STRIPE)])

    return pl.kernel(
        body,
        out_type=jax.ShapeDtypeStruct((NC, N_PAD, dh), jnp.float32),
        mesh=_mesh(),
        compiler_params=pltpu.CompilerParams(use_tc_tiling_on_sc=False),
        scratch_types=[
            pltpu.VMEM((E_TILE2,), jnp.int32),
            pltpu.VMEM((NCHUNK2, CHUNK), jnp.int32),
            pltpu.VMEM((NBUF, CHUNK, dh), jnp.float32),
            pltpu.VMEM_SHARED((N_PAD, dh), jnp.float32),
        ] + [pltpu.SemaphoreType.DMA] * NBUF,
    )(y_split, src1d, dst2d, zeros_stripe)


_ROW_BLK = 2048


def _dinv_block(deg_ref):
    deg = deg_ref[0, :, 0:1] + deg_ref[1, :, 0:1] + 1.0
    return lax.rsqrt(deg)


def _tc_y1(deg_parts, x_pad, w1):
    def body(deg_ref, x_ref, w_ref, y_ref):
        dinv = _dinv_block(deg_ref)
        y = jnp.dot(x_ref[...], w_ref[...], preferred_element_type=jnp.float32) * dinv
        y_ref[0] = y[:, : D_HID // 2]
        y_ref[1] = y[:, D_HID // 2 :]

    return pl.pallas_call(
        body,
        grid=(N_PAD // _ROW_BLK,),
        in_specs=[
            pl.BlockSpec((NC, _ROW_BLK, 16), lambda i: (0, i, 0)),
            pl.BlockSpec((_ROW_BLK, D_IN), lambda i: (i, 0)),
            pl.BlockSpec((D_IN, D_HID), lambda i: (0, 0)),
        ],
        out_specs=pl.BlockSpec((NC, _ROW_BLK, D_HID // 2), lambda i: (0, i, 0)),
        out_shape=jax.ShapeDtypeStruct((NC, N_PAD, D_HID // 2), jnp.float32),
    )(deg_parts, x_pad, w1)


def _tc_h_y2(deg_parts, acc1, y1, w2, b1):
    def body(deg_ref, acc_ref, y1_ref, w_ref, b_ref, y2_ref):
        dinv = _dinv_block(deg_ref)
        ridx = pl.program_id(0) * _ROW_BLK + lax.broadcasted_iota(
            jnp.int32, (_ROW_BLK, 1), 0
        )
        halves = []
        for c in range(NC):
            tot = acc_ref[c] + y1_ref[c]
            hc = jnp.maximum(tot * dinv + b_ref[c], 0.0)
            halves.append(jnp.where(ridx < N_NODES, hc, 0.0))
        h = jnp.concatenate(halves, axis=1)
        y2 = jnp.dot(h, w_ref[...], preferred_element_type=jnp.float32) * dinv
        y2_ref[0] = y2[:, : D_OUT // 2]
        y2_ref[1] = y2[:, D_OUT // 2 :]

    return pl.pallas_call(
        body,
        grid=(N_PAD // _ROW_BLK,),
        in_specs=[
            pl.BlockSpec((NC, _ROW_BLK, 16), lambda i: (0, i, 0)),
            pl.BlockSpec((NC, _ROW_BLK, D_HID // 2), lambda i: (0, i, 0)),
            pl.BlockSpec((NC, _ROW_BLK, D_HID // 2), lambda i: (0, i, 0)),
            pl.BlockSpec((D_HID, D_OUT), lambda i: (0, 0)),
            pl.BlockSpec((NC, 1, D_HID // 2), lambda i: (0, 0, 0)),
        ],
        out_specs=pl.BlockSpec((NC, _ROW_BLK, D_OUT // 2), lambda i: (0, i, 0)),
        out_shape=jax.ShapeDtypeStruct((NC, N_PAD, D_OUT // 2), jnp.float32),
    )(deg_parts, acc1, y1, w2, b1)


def _tc_out(deg_parts, acc2, y2, b2):
    def body(deg_ref, acc_ref, y2_ref, b_ref, o_ref):
        dinv = _dinv_block(deg_ref)
        halves = []
        for c in range(NC):
            tot = acc_ref[c] + y2_ref[c]
            halves.append(jnp.maximum(tot * dinv + b_ref[c], 0.0))
        o_ref[...] = jnp.concatenate(halves, axis=1)

    return pl.pallas_call(
        body,
        grid=(N_PAD // _ROW_BLK,),
        in_specs=[
            pl.BlockSpec((NC, _ROW_BLK, 16), lambda i: (0, i, 0)),
            pl.BlockSpec((NC, _ROW_BLK, D_OUT // 2), lambda i: (0, i, 0)),
            pl.BlockSpec((NC, _ROW_BLK, D_OUT // 2), lambda i: (0, i, 0)),
            pl.BlockSpec((NC, 1, D_OUT // 2), lambda i: (0, 0, 0)),
        ],
        out_specs=pl.BlockSpec((_ROW_BLK, D_OUT), lambda i: (i, 0)),
        out_shape=jax.ShapeDtypeStruct((N_PAD, D_OUT), jnp.float32),
    )(deg_parts, acc2, y2, b2)


def kernel(x, edge_index, W1, b1, W2, b2):
    src = edge_index[0].astype(jnp.int32)
    dst = edge_index[1].astype(jnp.int32)
    pad = jnp.full((E_PAD - E,), N_NODES, jnp.int32)
    src_pad = jnp.concatenate([src, pad])
    dst2d = jnp.concatenate([dst, pad]).reshape(E_PAD // CHUNK, CHUNK)
    x_pad = jnp.concatenate(
        [x, jnp.zeros((N_PAD - N_NODES, x.shape[1]), x.dtype)]
    )
    ones16 = jnp.ones((CHUNK, 16), jnp.float32)
    z16 = jnp.zeros((STRIPE, 16), jnp.float32)
    z64 = jnp.zeros((STRIPE, D_HID // 2), jnp.float32)
    z32 = jnp.zeros((STRIPE, D_OUT // 2), jnp.float32)

    deg_parts = _sc_degree(dst2d, ones16, z16)
    y1 = _tc_y1(deg_parts, x_pad, W1)
    acc1 = _sc_aggregate(y1, src_pad, dst2d, z64, D_HID // 2)
    y2 = _tc_h_y2(deg_parts, acc1, y1, W2, b1.reshape(NC, 1, -1))
    acc2 = _sc_aggregate(y2, src_pad, dst2d, z32, D_OUT // 2)
    out = _tc_out(deg_parts, acc2, y2, b2.reshape(NC, 1, -1))
    return out[:N_NODES]
